# baseline (speedup 1.0000x reference)
import functools

import jax
import jax.numpy as jnp
from jax import lax
from jax.experimental import pallas as pl
from jax.experimental.pallas import tpu as pltpu
from jax.experimental.pallas import tpu_sc as plsc

N = 8192
C = 1024
D = 2048
E = 8
RT = 256
TB = 256
P = N * 2 + E * TB
NT = P // TB

NW = 32
TW = N // NW
CH = 64
CHC = 16

_MESH = dict(core_axis_name="c", subcore_axis_name="s", num_cores=2,
             num_subcores=16)


def _router_body(x_ref, rw_ref, info_ref, counts_ref, cnt):
    t = pl.program_id(0)

    @pl.when(t == 0)
    def _():
        cnt[...] = jnp.zeros_like(cnt)

    x = x_ref[...]
    logits = lax.dot_general(x, rw_ref[...], (((1,), (1,)), ((), ())),
                             preferred_element_type=jnp.float32)
    m = jnp.max(logits, axis=-1, keepdims=True)
    ex = jnp.exp(logits - m)
    probs = ex / jnp.sum(ex, axis=-1, keepdims=True)

    col = lax.broadcasted_iota(jnp.int32, (RT, E), 1)
    i1 = jnp.argmax(probs, axis=-1).astype(jnp.int32)
    oh1 = col == i1[:, None]
    p1 = jnp.sum(jnp.where(oh1, probs, 0.0), axis=-1)
    i2 = jnp.argmax(jnp.where(oh1, -1.0, probs), axis=-1).astype(jnp.int32)
    oh2 = col == i2[:, None]
    p2 = jnp.sum(jnp.where(oh2, probs, 0.0), axis=-1)
    s = p1 + p2
    wa = p1 / s
    wb = p2 / s

    col16 = lax.broadcasted_iota(jnp.int32, (RT, 16), 1)
    oh1f = (col16 == i1[:, None]).astype(jnp.float32)
    oh2f = (col16 == i2[:, None]).astype(jnp.float32)
    tri = (lax.broadcasted_iota(jnp.int32, (RT, RT), 0)
           > lax.broadcasted_iota(jnp.int32, (RT, RT), 1)).astype(jnp.float32)
    exc1 = lax.dot_general(tri, oh1f, (((1,), (0,)), ((), ())),
                           preferred_element_type=jnp.float32)
    exc2 = lax.dot_general(tri, oh2f, (((1,), (0,)), ((), ())),
                           preferred_element_type=jnp.float32)
    tot1 = jnp.sum(oh1f, axis=0, keepdims=True)
    tot2 = jnp.sum(oh2f, axis=0, keepdims=True)
    base = cnt[...]
    r1 = jnp.sum(oh1f * (base + exc1), axis=-1)
    r2 = jnp.sum(oh2f * (base + tot1 + exc2), axis=-1)
    cnt[...] = base + tot1 + tot2
    counts_ref[...] = cnt[...]

    info_ref[...] = (jnp.where(col == 0, i1.astype(jnp.float32)[:, None], 0.0)
                     + jnp.where(col == 1, i2.astype(jnp.float32)[:, None], 0.0)
                     + jnp.where(col == 2, wa[:, None], 0.0)
                     + jnp.where(col == 3, wb[:, None], 0.0)
                     + jnp.where(col == 4, r1[:, None], 0.0)
                     + jnp.where(col == 5, r2[:, None], 0.0))


def _router(x_flat, router_w):
    return pl.pallas_call(
        _router_body,
        grid=(N // RT,),
        in_specs=[
            pl.BlockSpec((RT, C), lambda t: (t, 0)),
            pl.BlockSpec((E, C), lambda t: (0, 0)),
        ],
        out_specs=[
            pl.BlockSpec((RT, E), lambda t: (t, 0)),
            pl.BlockSpec((1, 16), lambda t: (0, 0)),
        ],
        out_shape=[
            jax.ShapeDtypeStruct((N, E), jnp.float32),
            jax.ShapeDtypeStruct((1, 16), jnp.float32),
        ],
        scratch_shapes=[pltpu.VMEM((1, 16), jnp.float32)],
    )(x_flat, router_w)


def _load_starts(counts_hbm, counts_v, starts_tab):
    pltpu.sync_copy(counts_hbm, counts_v)
    cnt = counts_v[0].astype(jnp.int32)
    padded = ((cnt + (TB - 1)) >> 8) << 8
    starts_tab[...] = plsc.cumsum(padded) - padded


def _slot_positions(info_v, starts_tab, v):
    rows = lax.iota(jnp.int32, 16) + v * 16
    z = jnp.zeros((16,), jnp.int32)
    i1 = plsc.load_gather(info_v, [rows, z]).astype(jnp.int32)
    i2 = plsc.load_gather(info_v, [rows, z + 1]).astype(jnp.int32)
    r1 = plsc.load_gather(info_v, [rows, z + 4]).astype(jnp.int32)
    r2 = plsc.load_gather(info_v, [rows, z + 5]).astype(jnp.int32)
    pa = plsc.load_gather(starts_tab, [i1]) + r1
    pb = plsc.load_gather(starts_tab, [i2]) + r2
    return rows, z, pa, pb


def _scatter_body(x_hbm, info_hbm, counts_hbm, xs_hbm,
                  counts_v, starts_tab, info_v, x_v, posa, posb, sem):
    wid = lax.axis_index("c") * 16 + lax.axis_index("s")
    _load_starts(counts_hbm, counts_v, starts_tab)
    for c in range(TW // CH):
        base = wid * TW + c * CH
        pltpu.sync_copy(info_hbm.at[pl.ds(base, CH)], info_v)
        pltpu.sync_copy(x_hbm.at[pl.ds(base, CH)], x_v)
        for v in range(CH // 16):
            _, _, pa, pb = _slot_positions(info_v, starts_tab, v)
            posa[pl.ds(v * 16, 16)] = pa
            posb[pl.ds(v * 16, 16)] = pb
        cps = [pltpu.async_copy(x_v, xs_hbm.at[posa], sem),
               pltpu.async_copy(x_v, xs_hbm.at[posb], sem)]
        for cp in cps:
            cp.wait()


def _scatter(x_flat, info, counts):
    return pl.kernel(
        _scatter_body,
        out_type=jax.ShapeDtypeStruct((P, C), jnp.float32),
        mesh=plsc.VectorSubcoreMesh(**_MESH),
        compiler_params=pltpu.CompilerParams(needs_layout_passes=False),
        scratch_types=[
            pltpu.VMEM((1, 16), jnp.float32),
            pltpu.VMEM((16,), jnp.int32),
            pltpu.VMEM((CH, E), jnp.float32),
            pltpu.VMEM((CH, C), jnp.float32),
            pltpu.VMEM((CH,), jnp.int32),
            pltpu.VMEM((CH,), jnp.int32),
            pltpu.SemaphoreType.DMA,
        ],
    )(x_flat, info, counts)


def _moe_body(te_ref, xs_ref, w1_ref, w3_ref, w2_ref, ys_ref):
    t = pl.program_id(0)

    @pl.when(te_ref[t] < E)
    def _():
        x = xs_ref[...]
        a = lax.dot_general(x, w1_ref[0], (((1,), (1,)), ((), ())),
                            preferred_element_type=jnp.float32)
        b = lax.dot_general(x, w3_ref[0], (((1,), (1,)), ((), ())),
                            preferred_element_type=jnp.float32)
        h = (a * jax.nn.sigmoid(a)) * b
        y = lax.dot_general(h, w2_ref[0], (((1,), (1,)), ((), ())),
                            preferred_element_type=jnp.float32)
        ys_ref[...] = y


def _moe(tile_expert, xs, w1, w3, w2):
    def _wmap(t, te):
        return (jnp.minimum(te[t], E - 1), 0, 0)

    grid_spec = pltpu.PrefetchScalarGridSpec(
        num_scalar_prefetch=1,
        grid=(NT,),
        in_specs=[
            pl.BlockSpec((TB, C), lambda t, te: (t, 0)),
            pl.BlockSpec((1, D, C), _wmap),
            pl.BlockSpec((1, D, C), _wmap),
            pl.BlockSpec((1, C, D), _wmap),
        ],
        out_specs=pl.BlockSpec((TB, C), lambda t, te: (t, 0)),
    )
    return pl.pallas_call(
        _moe_body,
        grid_spec=grid_spec,
        out_shape=jax.ShapeDtypeStruct((P, C), jnp.float32),
    )(tile_expert, xs, w1, w3, w2)


def _combine_body(ys_hbm, info_hbm, counts_hbm, out_hbm,
                  counts_v, starts_tab, info_v, posa, posb, y1, y2, yo,
                  semg, semo):
    wid = lax.axis_index("c") * 16 + lax.axis_index("s")
    _load_starts(counts_hbm, counts_v, starts_tab)
    nch = TW // CHC

    def stage(s, c):
        base = wid * TW + c * CHC
        pltpu.sync_copy(info_hbm.at[pl.ds(base, CHC)], info_v[s])
        _, _, pa, pb = _slot_positions(info_v[s], starts_tab, 0)
        posa[s][...] = pa
        posb[s][...] = pb
        pltpu.async_copy(ys_hbm.at[posa[s]], y1[s], semg[s])
        pltpu.async_copy(ys_hbm.at[posb[s]], y2[s], semg[s])

    def process(s, c):
        base = wid * TW + c * CHC
        pltpu.make_async_copy(ys_hbm.at[posa[s]], y1[s], semg[s]).wait()
        pltpu.make_async_copy(ys_hbm.at[posb[s]], y2[s], semg[s]).wait()

        @pl.loop(0, CHC)
        def _(r):
            ridx = jnp.zeros((16,), jnp.int32) + r
            wa = plsc.load_gather(info_v[s], [ridx, ridx * 0 + 2])
            wb = plsc.load_gather(info_v[s], [ridx, ridx * 0 + 3])
            for k in range(C // 16):
                sl = pl.ds(k * 16, 16)
                yo[s][r, sl] = y1[s][r, sl] * wa + y2[s][r, sl] * wb

        pltpu.async_copy(yo[s], out_hbm.at[pl.ds(base, CHC)], semo[s])

    stage(0, 0)

    @pl.loop(0, nch // 2)
    def _(p):
        c0 = 2 * p

        @pl.when(p > 0)
        def _():
            pltpu.make_async_copy(yo[1], out_hbm.at[pl.ds(0, CHC)],
                                  semo[1]).wait()

        stage(1, c0 + 1)
        process(0, c0)

        @pl.when(c0 + 2 < nch)
        def _():
            pltpu.make_async_copy(yo[0], out_hbm.at[pl.ds(0, CHC)],
                                  semo[0]).wait()
            stage(0, c0 + 2)

        process(1, c0 + 1)

    pltpu.make_async_copy(yo[0], out_hbm.at[pl.ds(0, CHC)], semo[0]).wait()
    pltpu.make_async_copy(yo[1], out_hbm.at[pl.ds(0, CHC)], semo[1]).wait()


def _combine(ys, info, counts):
    def buf(shape, dtype):
        return [pltpu.VMEM(shape, dtype) for _ in range(2)]

    return pl.kernel(
        _combine_body,
        out_type=jax.ShapeDtypeStruct((N, C), jnp.float32),
        mesh=plsc.VectorSubcoreMesh(**_MESH),
        compiler_params=pltpu.CompilerParams(needs_layout_passes=False),
        scratch_types=[
            pltpu.VMEM((1, 16), jnp.float32),
            pltpu.VMEM((16,), jnp.int32),
            buf((CHC, E), jnp.float32),
            buf((CHC,), jnp.int32),
            buf((CHC,), jnp.int32),
            buf((CHC, C), jnp.float32),
            buf((CHC, C), jnp.float32),
            buf((CHC, C), jnp.float32),
            [pltpu.SemaphoreType.DMA for _ in range(2)],
            [pltpu.SemaphoreType.DMA for _ in range(2)],
        ],
    )(ys, info, counts)


def kernel(x, router_w, w1, w2, w3):
    B, T, Cc = x.shape
    x_flat = x.reshape(N, C)

    info, counts_f = _router(x_flat, router_w)

    counts = counts_f[0, :E].astype(jnp.int32)
    padded = ((counts + TB - 1) // TB) * TB
    ends = jnp.cumsum(padded)
    tile_expert = jnp.searchsorted(
        ends, jnp.arange(NT, dtype=jnp.int32) * TB,
        side="right").astype(jnp.int32)

    xs = _scatter(x_flat, info, counts_f)
    ys = _moe(tile_expert, xs, w1, w3, w2)
    out = _combine(ys, info, counts_f)
    return out.reshape(B, T, Cc)


# speedup vs baseline: 1.0211x; 1.0211x over previous
import functools

import jax
import jax.numpy as jnp
from jax import lax
from jax.experimental import pallas as pl
from jax.experimental.pallas import tpu as pltpu
from jax.experimental.pallas import tpu_sc as plsc

N = 8192
C = 1024
D = 2048
E = 8
RT = 512
TB = 256
P = N * 2 + E * TB
NT = P // TB

NW = 32
TW = N // NW
CH = 64
CHC = 16

_MESH = dict(core_axis_name="c", subcore_axis_name="s", num_cores=2,
             num_subcores=16)


def _router_body(x_ref, rw_ref, info_ref, counts_ref, cnt):
    t = pl.program_id(0)

    @pl.when(t == 0)
    def _():
        cnt[...] = jnp.zeros_like(cnt)

    x = x_ref[...]
    logits = lax.dot_general(x, rw_ref[...], (((1,), (1,)), ((), ())),
                             preferred_element_type=jnp.float32)
    m = jnp.max(logits, axis=-1, keepdims=True)
    ex = jnp.exp(logits - m)
    probs = ex / jnp.sum(ex, axis=-1, keepdims=True)

    col = lax.broadcasted_iota(jnp.int32, (RT, E), 1)
    i1 = jnp.argmax(probs, axis=-1).astype(jnp.int32)
    oh1 = col == i1[:, None]
    p1 = jnp.sum(jnp.where(oh1, probs, 0.0), axis=-1)
    i2 = jnp.argmax(jnp.where(oh1, -1.0, probs), axis=-1).astype(jnp.int32)
    oh2 = col == i2[:, None]
    p2 = jnp.sum(jnp.where(oh2, probs, 0.0), axis=-1)
    s = p1 + p2
    wa = p1 / s
    wb = p2 / s

    col16 = lax.broadcasted_iota(jnp.int32, (RT, 16), 1)
    oh1f = (col16 == i1[:, None]).astype(jnp.float32)
    oh2f = (col16 == i2[:, None]).astype(jnp.float32)
    tri = (lax.broadcasted_iota(jnp.int32, (RT, RT), 0)
           > lax.broadcasted_iota(jnp.int32, (RT, RT), 1)).astype(jnp.float32)
    exc1 = lax.dot_general(tri, oh1f, (((1,), (0,)), ((), ())),
                           preferred_element_type=jnp.float32)
    exc2 = lax.dot_general(tri, oh2f, (((1,), (0,)), ((), ())),
                           preferred_element_type=jnp.float32)
    tot1 = jnp.sum(oh1f, axis=0, keepdims=True)
    tot2 = jnp.sum(oh2f, axis=0, keepdims=True)
    base = cnt[...]
    r1 = jnp.sum(oh1f * (base + exc1), axis=-1)
    r2 = jnp.sum(oh2f * (base + tot1 + exc2), axis=-1)
    cnt[...] = base + tot1 + tot2
    counts_ref[...] = cnt[...]

    info_ref[...] = (jnp.where(col == 0, i1.astype(jnp.float32)[:, None], 0.0)
                     + jnp.where(col == 1, i2.astype(jnp.float32)[:, None], 0.0)
                     + jnp.where(col == 2, wa[:, None], 0.0)
                     + jnp.where(col == 3, wb[:, None], 0.0)
                     + jnp.where(col == 4, r1[:, None], 0.0)
                     + jnp.where(col == 5, r2[:, None], 0.0))


def _router(x_flat, router_w):
    return pl.pallas_call(
        _router_body,
        grid=(N // RT,),
        in_specs=[
            pl.BlockSpec((RT, C), lambda t: (t, 0)),
            pl.BlockSpec((E, C), lambda t: (0, 0)),
        ],
        out_specs=[
            pl.BlockSpec((RT, E), lambda t: (t, 0)),
            pl.BlockSpec((1, 16), lambda t: (0, 0)),
        ],
        out_shape=[
            jax.ShapeDtypeStruct((N, E), jnp.float32),
            jax.ShapeDtypeStruct((1, 16), jnp.float32),
        ],
        scratch_shapes=[pltpu.VMEM((1, 16), jnp.float32)],
    )(x_flat, router_w)


def _load_starts(counts_hbm, counts_v, starts_tab):
    pltpu.sync_copy(counts_hbm, counts_v)
    cnt = counts_v[0].astype(jnp.int32)
    padded = ((cnt + (TB - 1)) >> 8) << 8
    starts_tab[...] = plsc.cumsum(padded) - padded


def _slot_positions(info_v, starts_tab, v):
    rows = lax.iota(jnp.int32, 16) + v * 16
    z = jnp.zeros((16,), jnp.int32)
    i1 = plsc.load_gather(info_v, [rows, z]).astype(jnp.int32)
    i2 = plsc.load_gather(info_v, [rows, z + 1]).astype(jnp.int32)
    r1 = plsc.load_gather(info_v, [rows, z + 4]).astype(jnp.int32)
    r2 = plsc.load_gather(info_v, [rows, z + 5]).astype(jnp.int32)
    pa = plsc.load_gather(starts_tab, [i1]) + r1
    pb = plsc.load_gather(starts_tab, [i2]) + r2
    return rows, z, pa, pb


def _scatter_body(x_hbm, info_hbm, counts_hbm, xs_hbm,
                  counts_v, starts_tab, info_v, x_v, posa, posb, sem):
    wid = lax.axis_index("c") * 16 + lax.axis_index("s")
    _load_starts(counts_hbm, counts_v, starts_tab)
    for c in range(TW // CH):
        base = wid * TW + c * CH
        pltpu.sync_copy(info_hbm.at[pl.ds(base, CH)], info_v)
        pltpu.sync_copy(x_hbm.at[pl.ds(base, CH)], x_v)
        for v in range(CH // 16):
            _, _, pa, pb = _slot_positions(info_v, starts_tab, v)
            posa[pl.ds(v * 16, 16)] = pa
            posb[pl.ds(v * 16, 16)] = pb
        cps = [pltpu.async_copy(x_v, xs_hbm.at[posa], sem),
               pltpu.async_copy(x_v, xs_hbm.at[posb], sem)]
        for cp in cps:
            cp.wait()


def _scatter(x_flat, info, counts):
    return pl.kernel(
        _scatter_body,
        out_type=jax.ShapeDtypeStruct((P, C), jnp.float32),
        mesh=plsc.VectorSubcoreMesh(**_MESH),
        compiler_params=pltpu.CompilerParams(needs_layout_passes=False),
        scratch_types=[
            pltpu.VMEM((1, 16), jnp.float32),
            pltpu.VMEM((16,), jnp.int32),
            pltpu.VMEM((CH, E), jnp.float32),
            pltpu.VMEM((CH, C), jnp.float32),
            pltpu.VMEM((CH,), jnp.int32),
            pltpu.VMEM((CH,), jnp.int32),
            pltpu.SemaphoreType.DMA,
        ],
    )(x_flat, info, counts)


def _moe_body(te_ref, xs_ref, w1_ref, w3_ref, w2_ref, ys_ref):
    t = pl.program_id(0)

    @pl.when(te_ref[t] < E)
    def _():
        x = xs_ref[...]
        a = lax.dot_general(x, w1_ref[0], (((1,), (1,)), ((), ())),
                            preferred_element_type=jnp.float32)
        b = lax.dot_general(x, w3_ref[0], (((1,), (1,)), ((), ())),
                            preferred_element_type=jnp.float32)
        h = (a * jax.nn.sigmoid(a)) * b
        y = lax.dot_general(h, w2_ref[0], (((1,), (1,)), ((), ())),
                            preferred_element_type=jnp.float32)
        ys_ref[...] = y


def _moe(tile_expert, xs, w1, w3, w2):
    def _wmap(t, te):
        return (jnp.minimum(te[t], E - 1), 0, 0)

    grid_spec = pltpu.PrefetchScalarGridSpec(
        num_scalar_prefetch=1,
        grid=(NT,),
        in_specs=[
            pl.BlockSpec((TB, C), lambda t, te: (t, 0)),
            pl.BlockSpec((1, D, C), _wmap),
            pl.BlockSpec((1, D, C), _wmap),
            pl.BlockSpec((1, C, D), _wmap),
        ],
        out_specs=pl.BlockSpec((TB, C), lambda t, te: (t, 0)),
    )
    return pl.pallas_call(
        _moe_body,
        grid_spec=grid_spec,
        out_shape=jax.ShapeDtypeStruct((P, C), jnp.float32),
    )(tile_expert, xs, w1, w3, w2)


def _combine_body(ys_hbm, info_hbm, counts_hbm, out_hbm,
                  counts_v, starts_tab, info_v, posa, posb, y1, y2, yo,
                  semg, semo):
    wid = lax.axis_index("c") * 16 + lax.axis_index("s")
    _load_starts(counts_hbm, counts_v, starts_tab)
    nch = TW // CHC

    def stage(s, c):
        base = wid * TW + c * CHC
        pltpu.sync_copy(info_hbm.at[pl.ds(base, CHC)], info_v[s])
        _, _, pa, pb = _slot_positions(info_v[s], starts_tab, 0)
        posa[s][...] = pa
        posb[s][...] = pb
        pltpu.async_copy(ys_hbm.at[posa[s]], y1[s], semg[s])
        pltpu.async_copy(ys_hbm.at[posb[s]], y2[s], semg[s])

    def process(s, c):
        base = wid * TW + c * CHC
        pltpu.make_async_copy(ys_hbm.at[posa[s]], y1[s], semg[s]).wait()
        pltpu.make_async_copy(ys_hbm.at[posb[s]], y2[s], semg[s]).wait()

        @pl.loop(0, CHC)
        def _(r):
            ridx = jnp.zeros((16,), jnp.int32) + r
            wa = plsc.load_gather(info_v[s], [ridx, ridx * 0 + 2])
            wb = plsc.load_gather(info_v[s], [ridx, ridx * 0 + 3])
            for k in range(C // 16):
                sl = pl.ds(k * 16, 16)
                yo[s][r, sl] = y1[s][r, sl] * wa + y2[s][r, sl] * wb

        pltpu.async_copy(yo[s], out_hbm.at[pl.ds(base, CHC)], semo[s])

    stage(0, 0)

    @pl.loop(0, nch // 2)
    def _(p):
        c0 = 2 * p

        @pl.when(p > 0)
        def _():
            pltpu.make_async_copy(yo[1], out_hbm.at[pl.ds(0, CHC)],
                                  semo[1]).wait()

        stage(1, c0 + 1)
        process(0, c0)

        @pl.when(c0 + 2 < nch)
        def _():
            pltpu.make_async_copy(yo[0], out_hbm.at[pl.ds(0, CHC)],
                                  semo[0]).wait()
            stage(0, c0 + 2)

        process(1, c0 + 1)

    pltpu.make_async_copy(yo[0], out_hbm.at[pl.ds(0, CHC)], semo[0]).wait()
    pltpu.make_async_copy(yo[1], out_hbm.at[pl.ds(0, CHC)], semo[1]).wait()


def _combine(ys, info, counts):
    def buf(shape, dtype):
        return [pltpu.VMEM(shape, dtype) for _ in range(2)]

    return pl.kernel(
        _combine_body,
        out_type=jax.ShapeDtypeStruct((N, C), jnp.float32),
        mesh=plsc.VectorSubcoreMesh(**_MESH),
        compiler_params=pltpu.CompilerParams(needs_layout_passes=False),
        scratch_types=[
            pltpu.VMEM((1, 16), jnp.float32),
            pltpu.VMEM((16,), jnp.int32),
            buf((CHC, E), jnp.float32),
            buf((CHC,), jnp.int32),
            buf((CHC,), jnp.int32),
            buf((CHC, C), jnp.float32),
            buf((CHC, C), jnp.float32),
            buf((CHC, C), jnp.float32),
            [pltpu.SemaphoreType.DMA for _ in range(2)],
            [pltpu.SemaphoreType.DMA for _ in range(2)],
        ],
    )(ys, info, counts)


def kernel(x, router_w, w1, w2, w3):
    B, T, Cc = x.shape
    x_flat = x.reshape(N, C)

    info, counts_f = _router(x_flat, router_w)

    counts = counts_f[0, :E].astype(jnp.int32)
    padded = ((counts + TB - 1) // TB) * TB
    ends = jnp.cumsum(padded)
    tile_expert = jnp.searchsorted(
        ends, jnp.arange(NT, dtype=jnp.int32) * TB,
        side="right").astype(jnp.int32)

    xs = _scatter(x_flat, info, counts_f)
    ys = _moe(tile_expert, xs, w1, w3, w2)
    out = _combine(ys, info, counts_f)
    return out.reshape(B, T, Cc)
